# baseline (speedup 1.0000x reference)
import functools

import jax
import jax.numpy as jnp
from jax import lax
from jax.experimental import pallas as pl
from jax.experimental.pallas import tpu as pltpu
from jax.experimental.pallas import tpu_sc as plsc

T = 2048
D = 2048
DE = 1024
E = 8
SH = 2048
BT = 128
MAXNT = 40
NP = MAXNT * BT
NCH = 32
CH = 128
NW = 32


def _plan_body(x_ref, wg_ref, pos_ref, w_ref, sinfo_ref,
               cnt_ref, eid_s, rank_s):
    c = pl.program_id(0)

    @pl.when(c == 0)
    def _():
        cnt_ref[...] = jnp.zeros_like(cnt_ref)

    x = x_ref[...]
    logits = lax.dot_general(x, wg_ref[...], (((1,), (1,)), ((), ())),
                             preferred_element_type=jnp.float32)
    m = jnp.max(logits, axis=-1, keepdims=True)
    ex = jnp.exp(logits - m)
    sm = ex / jnp.sum(ex, axis=-1, keepdims=True)
    lane = lax.broadcasted_iota(jnp.int32, (CH, E), 1)
    i0 = jnp.argmax(sm, axis=-1)
    one0 = lane == i0[:, None]
    sm1 = jnp.where(one0, -jnp.inf, sm)
    i1 = jnp.argmax(sm1, axis=-1)
    k_is_1 = c >= (NCH // 2)
    idx = jnp.where(k_is_1, i1, i0).astype(jnp.int32)
    w = jnp.where(k_is_1, jnp.max(sm1, axis=-1), jnp.max(sm, axis=-1))

    oh = (lane == idx[:, None]).astype(jnp.float32)
    r_i = lax.broadcasted_iota(jnp.int32, (CH, CH), 0)
    c_i = lax.broadcasted_iota(jnp.int32, (CH, CH), 1)
    tri = (r_i > c_i).astype(jnp.float32)
    within = lax.dot_general(tri, oh, (((1,), (0,)), ((), ())),
                             preferred_element_type=jnp.float32)
    base = cnt_ref[...]
    rank = jnp.sum((within + base) * oh, axis=1)

    eid_s[pl.ds(c, 1), :] = idx.reshape(1, CH)
    rank_s[pl.ds(c, 1), :] = rank.astype(jnp.int32).reshape(1, CH)
    w_ref[...] = w.reshape(1, 1, CH)
    cnt_ref[...] = base + jnp.sum(oh, axis=0, keepdims=True)

    @pl.when(c == NCH - 1)
    def _():
        cnt = cnt_ref[...]
        nt = jnp.floor((cnt + (BT - 1.0)) * (1.0 / BT))
        er = lax.broadcasted_iota(jnp.int32, (E, E), 0)
        ec = lax.broadcasted_iota(jnp.int32, (E, E), 1)
        mu = (er < ec).astype(jnp.float32)
        cum_excl = lax.dot_general(nt, mu, (((1,), (0,)), ((), ())),
                                   preferred_element_type=jnp.float32)
        cum_incl = cum_excl + nt
        gstart = (cum_excl * BT).astype(jnp.int32)
        ntot = jnp.sum(nt).astype(jnp.int32).reshape(1, 1)
        ti = lax.broadcasted_iota(jnp.int32, (48, E), 0)
        texp = jnp.sum((ti >= cum_incl.astype(jnp.int32)).astype(jnp.int32),
                       axis=1)
        texp = jnp.minimum(texp, E - 1).reshape(1, 48)
        sinfo_ref[...] = jnp.concatenate(
            [texp, ntot, jnp.zeros((1, 7), jnp.int32), gstart], axis=1)
        eids = eid_s[...]
        poss = rank_s[...]
        for e in range(E):
            ge = lax.slice(gstart, (0, e), (1, e + 1))
            poss = poss + jnp.where(eids == e, ge, 0)
        pos_ref[...] = poss.reshape(NCH, 1, CH)


def _plan(xf, Wg):
    return pl.pallas_call(
        _plan_body,
        grid=(NCH,),
        in_specs=[
            pl.BlockSpec((CH, D), lambda c: (lax.rem(c, NCH // 2), 0)),
            pl.BlockSpec((E, D), lambda c: (0, 0)),
        ],
        out_specs=[
            pl.BlockSpec((NCH, 1, CH), lambda c: (0, 0, 0)),
            pl.BlockSpec((1, 1, CH), lambda c: (c, 0, 0)),
            pl.BlockSpec((1, 64), lambda c: (0, 0)),
        ],
        out_shape=[
            jax.ShapeDtypeStruct((NCH, 1, CH), jnp.int32),
            jax.ShapeDtypeStruct((NCH, 1, CH), jnp.float32),
            jax.ShapeDtypeStruct((1, 64), jnp.int32),
        ],
        scratch_shapes=[
            pltpu.VMEM((1, E), jnp.float32),
            pltpu.VMEM((NCH, CH), jnp.int32),
            pltpu.VMEM((NCH, CH), jnp.int32),
        ],
        compiler_params=pltpu.CompilerParams(
            dimension_semantics=("arbitrary",)),
    )(xf, Wg)


@functools.cache
def _sc_mesh():
    return plsc.VectorSubcoreMesh(core_axis_name="c", subcore_axis_name="s",
                                  num_cores=2, num_subcores=16)


@functools.cache
def _sc_dispatch_kernel():
    return functools.partial(
        pl.kernel,
        out_type=(jax.ShapeDtypeStruct((NP, D), jnp.float32),
                  jax.ShapeDtypeStruct((NP,), jnp.float32)),
        mesh=_sc_mesh(),
        scratch_types=[
            pltpu.VMEM((CH,), jnp.int32),
            pltpu.VMEM((CH,), jnp.float32),
            pltpu.VMEM((16, D), jnp.float32),
            pltpu.VMEM((16, D), jnp.float32),
            pltpu.SemaphoreType.DMA,
            pltpu.SemaphoreType.DMA,
            pltpu.SemaphoreType.DMA,
            pltpu.SemaphoreType.DMA,
            pltpu.SemaphoreType.DMA,
        ],
    )(_sc_dispatch_body)


def _sc_dispatch_body(pos_h, w_h, x_h, xs_h, ws_h,
                      pos_v, w_v, rows_a, rows_b, sem_a, sem_b,
                      sem_oa, sem_ob, sem_w):
    wid = lax.axis_index("s") * 2 + lax.axis_index("c")
    sbase = wid * CH
    tbase = lax.rem(wid, 16) * CH
    pltpu.sync_copy(pos_h.at[pl.ds(sbase, CH)], pos_v)
    pltpu.sync_copy(w_h.at[pl.ds(sbase, CH)], w_v)
    cp_w = pltpu.async_copy(w_v, ws_h.at[pos_v], sem_w)
    bufs = (rows_a, rows_b)
    sin = (sem_a, sem_b)
    sout = (sem_oa, sem_ob)
    nch = CH // 16
    cps_in = [None, None]
    cps_out = [None, None]
    cps_in[0] = pltpu.async_copy(x_h.at[pl.ds(tbase, 16)], bufs[0], sin[0])
    for i in range(nch):
        cur = i % 2
        nxt = 1 - cur
        if i + 1 < nch:
            if cps_out[nxt] is not None:
                cps_out[nxt].wait()
            cps_in[nxt] = pltpu.async_copy(
                x_h.at[pl.ds(tbase + (i + 1) * 16, 16)], bufs[nxt], sin[nxt])
        cps_in[cur].wait()
        pv = pos_v[pl.ds(i * 16, 16)]
        cps_out[cur] = pltpu.async_copy(bufs[cur], xs_h.at[pv], sout[cur])
    cp_w.wait()
    cps_out[0].wait()
    cps_out[1].wait()


TPW = T // NW


@functools.cache
def _sc_combine_kernel():
    return functools.partial(
        pl.kernel,
        out_type=jax.ShapeDtypeStruct((T, D), jnp.float32),
        mesh=_sc_mesh(),
        scratch_types=[
            pltpu.VMEM((TPW,), jnp.int32),
            pltpu.VMEM((TPW,), jnp.int32),
            pltpu.VMEM((16, D), jnp.float32),
            pltpu.VMEM((16, D), jnp.float32),
            pltpu.VMEM((16, D), jnp.float32),
            pltpu.SemaphoreType.DMA,
            pltpu.SemaphoreType.DMA,
            pltpu.SemaphoreType.DMA,
        ],
    )(_sc_combine_body)


def _sc_combine_body(ys_h, sh_h, pos_h, out_h,
                     p0_v, p1_v, a_v, b_v, s_v, sem_a, sem_b, sem_o):
    wid = lax.axis_index("s") * 2 + lax.axis_index("c")
    tbase = wid * TPW
    pltpu.sync_copy(pos_h.at[pl.ds(tbase, TPW)], p0_v)
    pltpu.sync_copy(pos_h.at[pl.ds(T + tbase, TPW)], p1_v)
    nch = TPW // 16
    cp_a = pltpu.async_copy(ys_h.at[p0_v[pl.ds(0, 16)]], a_v, sem_a)
    cp_b = pltpu.async_copy(ys_h.at[p1_v[pl.ds(0, 16)]], b_v, sem_b)
    cp_o = None
    for cc in range(nch):
        cp_a.wait()
        cp_b.wait()
        if cp_o is not None:
            cp_o.wait()
        pltpu.sync_copy(sh_h.at[pl.ds(tbase + cc * 16, 16)], s_v)

        def _row(r, _):
            def _col(j, _):
                sl = pl.ds(j * 16, 16)
                s_v[r, sl] = a_v[r, sl] + b_v[r, sl] + s_v[r, sl]
                return 0

            return lax.fori_loop(0, D // 16, _col, 0)

        lax.fori_loop(0, 16, _row, 0)
        if cc + 1 < nch:
            cp_a = pltpu.async_copy(ys_h.at[p0_v[pl.ds((cc + 1) * 16, 16)]],
                                    a_v, sem_a)
            cp_b = pltpu.async_copy(ys_h.at[p1_v[pl.ds((cc + 1) * 16, 16)]],
                                    b_v, sem_b)
        cp_o = pltpu.async_copy(s_v, out_h.at[pl.ds(tbase + cc * 16, 16)],
                                sem_o)
    cp_o.wait()


def _gemm_body(sinfo_ref, xs_ref, ws_ref, wge_ref, wue_ref, wde_ref, ys_ref):
    i = pl.program_id(0)

    @pl.when(i < sinfo_ref[48])
    def _():
        xs = xs_ref[...]
        g = lax.dot_general(xs, wge_ref[0], (((1,), (1,)), ((), ())),
                            preferred_element_type=jnp.float32)
        u = lax.dot_general(xs, wue_ref[0], (((1,), (1,)), ((), ())),
                            preferred_element_type=jnp.float32)
        h = g * lax.logistic(g) * u * ws_ref[0]
        ys_ref[...] = lax.dot_general(h, wde_ref[0], (((1,), (1,)), ((), ())),
                                      preferred_element_type=jnp.float32)


def _gemm(sinfo, xs, ws, Wge, Wue, Wde):
    def _clamped(i, s):
        return (jnp.minimum(i, s[48] - 1), 0)

    def _clamped3(i, s):
        return (jnp.minimum(i, s[48] - 1), 0, 0)

    grid_spec = pltpu.PrefetchScalarGridSpec(
        num_scalar_prefetch=1,
        grid=(MAXNT,),
        in_specs=[
            pl.BlockSpec((BT, D), _clamped),
            pl.BlockSpec((1, BT, 1), _clamped3),
            pl.BlockSpec((1, DE, D), lambda i, s: (s[i], 0, 0)),
            pl.BlockSpec((1, DE, D), lambda i, s: (s[i], 0, 0)),
            pl.BlockSpec((1, D, DE), lambda i, s: (s[i], 0, 0)),
        ],
        out_specs=pl.BlockSpec((BT, D), _clamped),
    )
    return pl.pallas_call(
        _gemm_body,
        grid_spec=grid_spec,
        out_shape=jax.ShapeDtypeStruct((NP, D), jnp.float32),
        compiler_params=pltpu.CompilerParams(
            dimension_semantics=("arbitrary",)),
    )(sinfo, xs, ws.reshape(MAXNT, BT, 1), Wge, Wue, Wde)


def _shared_body(x_ref, wsg_ref, wsu_ref, wsd_ref, o_ref):
    k = pl.program_id(1)
    x = x_ref[...]
    g = lax.dot_general(x, wsg_ref[...], (((1,), (1,)), ((), ())),
                        preferred_element_type=jnp.float32)
    u = lax.dot_general(x, wsu_ref[...], (((1,), (1,)), ((), ())),
                        preferred_element_type=jnp.float32)
    h = g * lax.logistic(g) * u
    y = lax.dot_general(h, wsd_ref[...], (((1,), (1,)), ((), ())),
                        preferred_element_type=jnp.float32)

    @pl.when(k == 0)
    def _():
        o_ref[...] = y

    @pl.when(k != 0)
    def _():
        o_ref[...] += y


def _shared(xf, Wsg, Wsu, Wsd):
    RS, CS = 512, 512
    return pl.pallas_call(
        _shared_body,
        grid=(T // RS, SH // CS),
        in_specs=[
            pl.BlockSpec((RS, D), lambda i, k: (i, 0)),
            pl.BlockSpec((CS, D), lambda i, k: (k, 0)),
            pl.BlockSpec((CS, D), lambda i, k: (k, 0)),
            pl.BlockSpec((D, CS), lambda i, k: (0, k)),
        ],
        out_specs=pl.BlockSpec((RS, D), lambda i, k: (i, 0)),
        out_shape=jax.ShapeDtypeStruct((T, D), jnp.float32),
        compiler_params=pltpu.CompilerParams(
            dimension_semantics=("arbitrary", "arbitrary")),
    )(xf, Wsg, Wsu, Wsd)


def kernel(x, Wg, Wge, Wue, Wde, Wsg, Wsu, Wsd):
    b, s, d = x.shape
    xf = x.reshape(-1, d)

    pos, w, sinfo = _plan(xf, Wg)
    posf = pos.reshape(-1)
    wf = w.reshape(-1)
    sinfof = sinfo.reshape(-1)

    xs, ws = _sc_dispatch_kernel()(posf, wf, xf)
    shared = _shared(xf, Wsg, Wsu, Wsd)
    ys = _gemm(sinfof, xs, ws, Wge, Wue, Wde)
    out = _sc_combine_kernel()(ys, shared, posf)
    return out.reshape(b, s, d)


# speedup vs baseline: 1.0014x; 1.0014x over previous
import functools

import jax
import jax.numpy as jnp
from jax import lax
from jax.experimental import pallas as pl
from jax.experimental.pallas import tpu as pltpu
from jax.experimental.pallas import tpu_sc as plsc

T = 2048
D = 2048
DE = 1024
E = 8
SH = 2048
BT = 128
MAXNT = 40
NP = MAXNT * BT
NCH = 32
CH = 128
NW = 32


def _plan_body(x_ref, wg_ref, pos_ref, w_ref, sinfo_ref,
               cnt_ref, eid_s, rank_s):
    c = pl.program_id(0)

    @pl.when(c == 0)
    def _():
        cnt_ref[...] = jnp.zeros_like(cnt_ref)

    x = x_ref[...]
    logits = lax.dot_general(x, wg_ref[...], (((1,), (1,)), ((), ())),
                             preferred_element_type=jnp.float32)
    m = jnp.max(logits, axis=-1, keepdims=True)
    ex = jnp.exp(logits - m)
    sm = ex / jnp.sum(ex, axis=-1, keepdims=True)
    lane = lax.broadcasted_iota(jnp.int32, (CH, E), 1)
    i0 = jnp.argmax(sm, axis=-1)
    one0 = lane == i0[:, None]
    sm1 = jnp.where(one0, -jnp.inf, sm)
    i1 = jnp.argmax(sm1, axis=-1)
    k_is_1 = c >= (NCH // 2)
    idx = jnp.where(k_is_1, i1, i0).astype(jnp.int32)
    w = jnp.where(k_is_1, jnp.max(sm1, axis=-1), jnp.max(sm, axis=-1))

    oh = (lane == idx[:, None]).astype(jnp.float32)
    r_i = lax.broadcasted_iota(jnp.int32, (CH, CH), 0)
    c_i = lax.broadcasted_iota(jnp.int32, (CH, CH), 1)
    tri = (r_i > c_i).astype(jnp.float32)
    within = lax.dot_general(tri, oh, (((1,), (0,)), ((), ())),
                             preferred_element_type=jnp.float32)
    base = cnt_ref[...]
    rank = jnp.sum((within + base) * oh, axis=1)

    eid_s[pl.ds(c, 1), :] = idx.reshape(1, CH)
    rank_s[pl.ds(c, 1), :] = rank.astype(jnp.int32).reshape(1, CH)
    w_ref[...] = w.reshape(1, 1, CH)
    cnt_ref[...] = base + jnp.sum(oh, axis=0, keepdims=True)

    @pl.when(c == NCH - 1)
    def _():
        cnt = cnt_ref[...]
        nt = jnp.floor((cnt + (BT - 1.0)) * (1.0 / BT))
        er = lax.broadcasted_iota(jnp.int32, (E, E), 0)
        ec = lax.broadcasted_iota(jnp.int32, (E, E), 1)
        mu = (er < ec).astype(jnp.float32)
        cum_excl = lax.dot_general(nt, mu, (((1,), (0,)), ((), ())),
                                   preferred_element_type=jnp.float32)
        cum_incl = cum_excl + nt
        gstart = (cum_excl * BT).astype(jnp.int32)
        ntot = jnp.sum(nt).astype(jnp.int32).reshape(1, 1)
        ti = lax.broadcasted_iota(jnp.int32, (48, E), 0)
        texp = jnp.sum((ti >= cum_incl.astype(jnp.int32)).astype(jnp.int32),
                       axis=1)
        texp = jnp.minimum(texp, E - 1).reshape(1, 48)
        sinfo_ref[...] = jnp.concatenate(
            [texp, ntot, jnp.zeros((1, 7), jnp.int32), gstart], axis=1)
        eids = eid_s[...]
        poss = rank_s[...]
        for e in range(E):
            ge = lax.slice(gstart, (0, e), (1, e + 1))
            poss = poss + jnp.where(eids == e, ge, 0)
        pos_ref[...] = poss.reshape(NCH, 1, CH)


def _plan(xf, Wg):
    return pl.pallas_call(
        _plan_body,
        grid=(NCH,),
        in_specs=[
            pl.BlockSpec((CH, D), lambda c: (lax.rem(c, NCH // 2), 0)),
            pl.BlockSpec((E, D), lambda c: (0, 0)),
        ],
        out_specs=[
            pl.BlockSpec((NCH, 1, CH), lambda c: (0, 0, 0)),
            pl.BlockSpec((1, 1, CH), lambda c: (c, 0, 0)),
            pl.BlockSpec((1, 64), lambda c: (0, 0)),
        ],
        out_shape=[
            jax.ShapeDtypeStruct((NCH, 1, CH), jnp.int32),
            jax.ShapeDtypeStruct((NCH, 1, CH), jnp.float32),
            jax.ShapeDtypeStruct((1, 64), jnp.int32),
        ],
        scratch_shapes=[
            pltpu.VMEM((1, E), jnp.float32),
            pltpu.VMEM((NCH, CH), jnp.int32),
            pltpu.VMEM((NCH, CH), jnp.int32),
        ],
        compiler_params=pltpu.CompilerParams(
            dimension_semantics=("arbitrary",)),
    )(xf, Wg)


@functools.cache
def _sc_mesh():
    return plsc.VectorSubcoreMesh(core_axis_name="c", subcore_axis_name="s",
                                  num_cores=2, num_subcores=16)


@functools.cache
def _sc_dispatch_kernel():
    return functools.partial(
        pl.kernel,
        out_type=(jax.ShapeDtypeStruct((NP, D), jnp.float32),
                  jax.ShapeDtypeStruct((NP,), jnp.float32)),
        mesh=_sc_mesh(),
        scratch_types=[
            pltpu.VMEM((CH,), jnp.int32),
            pltpu.VMEM((CH,), jnp.float32),
            pltpu.VMEM((16, D), jnp.float32),
            pltpu.VMEM((16, D), jnp.float32),
            pltpu.SemaphoreType.DMA,
            pltpu.SemaphoreType.DMA,
            pltpu.SemaphoreType.DMA,
            pltpu.SemaphoreType.DMA,
            pltpu.SemaphoreType.DMA,
        ],
    )(_sc_dispatch_body)


def _sc_dispatch_body(pos_h, w_h, x_h, xs_h, ws_h,
                      pos_v, w_v, rows_a, rows_b, sem_a, sem_b,
                      sem_oa, sem_ob, sem_w):
    wid = lax.axis_index("s") * 2 + lax.axis_index("c")
    sbase = wid * CH
    tbase = lax.rem(wid, 16) * CH
    pltpu.sync_copy(pos_h.at[pl.ds(sbase, CH)], pos_v)
    pltpu.sync_copy(w_h.at[pl.ds(sbase, CH)], w_v)
    cp_w = pltpu.async_copy(w_v, ws_h.at[pos_v], sem_w)
    bufs = (rows_a, rows_b)
    sin = (sem_a, sem_b)
    sout = (sem_oa, sem_ob)
    nch = CH // 16
    cps_in = [None, None]
    cps_out = [None, None]
    cps_in[0] = pltpu.async_copy(x_h.at[pl.ds(tbase, 16)], bufs[0], sin[0])
    for i in range(nch):
        cur = i % 2
        nxt = 1 - cur
        if i + 1 < nch:
            if cps_out[nxt] is not None:
                cps_out[nxt].wait()
            cps_in[nxt] = pltpu.async_copy(
                x_h.at[pl.ds(tbase + (i + 1) * 16, 16)], bufs[nxt], sin[nxt])
        cps_in[cur].wait()
        pv = pos_v[pl.ds(i * 16, 16)]
        cps_out[cur] = pltpu.async_copy(bufs[cur], xs_h.at[pv], sout[cur])
    cp_w.wait()
    cps_out[0].wait()
    cps_out[1].wait()


TPW = T // NW


@functools.cache
def _sc_combine_kernel():
    return functools.partial(
        pl.kernel,
        out_type=jax.ShapeDtypeStruct((T, D), jnp.float32),
        mesh=_sc_mesh(),
        scratch_types=[
            pltpu.VMEM((TPW,), jnp.int32),
            pltpu.VMEM((TPW,), jnp.int32),
            pltpu.VMEM((16, D), jnp.float32),
            pltpu.VMEM((16, D), jnp.float32),
            pltpu.VMEM((16, D), jnp.float32),
            pltpu.SemaphoreType.DMA,
            pltpu.SemaphoreType.DMA,
            pltpu.SemaphoreType.DMA,
        ],
    )(_sc_combine_body)


def _sc_combine_body(ys_h, sh_h, pos_h, out_h,
                     p0_v, p1_v, a_v, b_v, s_v, sem_a, sem_b, sem_o):
    wid = lax.axis_index("s") * 2 + lax.axis_index("c")
    tbase = wid * TPW
    pltpu.sync_copy(pos_h.at[pl.ds(tbase, TPW)], p0_v)
    pltpu.sync_copy(pos_h.at[pl.ds(T + tbase, TPW)], p1_v)
    nch = TPW // 16
    cp_a = pltpu.async_copy(ys_h.at[p0_v[pl.ds(0, 16)]], a_v, sem_a)
    cp_b = pltpu.async_copy(ys_h.at[p1_v[pl.ds(0, 16)]], b_v, sem_b)
    cp_o = None
    for cc in range(nch):
        cp_a.wait()
        cp_b.wait()
        if cp_o is not None:
            cp_o.wait()
        pltpu.sync_copy(sh_h.at[pl.ds(tbase + cc * 16, 16)], s_v)

        for r in range(16):
            def _col(j, _, r=r):
                for u in range(4):
                    sl = pl.ds(j * 64 + u * 16, 16)
                    s_v[r, sl] = a_v[r, sl] + b_v[r, sl] + s_v[r, sl]
                return 0

            lax.fori_loop(0, D // 64, _col, 0)
        if cc + 1 < nch:
            cp_a = pltpu.async_copy(ys_h.at[p0_v[pl.ds((cc + 1) * 16, 16)]],
                                    a_v, sem_a)
            cp_b = pltpu.async_copy(ys_h.at[p1_v[pl.ds((cc + 1) * 16, 16)]],
                                    b_v, sem_b)
        cp_o = pltpu.async_copy(s_v, out_h.at[pl.ds(tbase + cc * 16, 16)],
                                sem_o)
    cp_o.wait()


def _gemm_body(sinfo_ref, xs_ref, ws_ref, wge_ref, wue_ref, wde_ref, ys_ref):
    i = pl.program_id(0)

    @pl.when(i < sinfo_ref[48])
    def _():
        xs = xs_ref[...]
        g = lax.dot_general(xs, wge_ref[0], (((1,), (1,)), ((), ())),
                            preferred_element_type=jnp.float32)
        u = lax.dot_general(xs, wue_ref[0], (((1,), (1,)), ((), ())),
                            preferred_element_type=jnp.float32)
        h = g * lax.logistic(g) * u * ws_ref[0]
        ys_ref[...] = lax.dot_general(h, wde_ref[0], (((1,), (1,)), ((), ())),
                                      preferred_element_type=jnp.float32)


def _gemm(sinfo, xs, ws, Wge, Wue, Wde):
    def _clamped(i, s):
        return (jnp.minimum(i, s[48] - 1), 0)

    def _clamped3(i, s):
        return (jnp.minimum(i, s[48] - 1), 0, 0)

    grid_spec = pltpu.PrefetchScalarGridSpec(
        num_scalar_prefetch=1,
        grid=(MAXNT,),
        in_specs=[
            pl.BlockSpec((BT, D), _clamped),
            pl.BlockSpec((1, BT, 1), _clamped3),
            pl.BlockSpec((1, DE, D), lambda i, s: (s[i], 0, 0)),
            pl.BlockSpec((1, DE, D), lambda i, s: (s[i], 0, 0)),
            pl.BlockSpec((1, D, DE), lambda i, s: (s[i], 0, 0)),
        ],
        out_specs=pl.BlockSpec((BT, D), _clamped),
    )
    return pl.pallas_call(
        _gemm_body,
        grid_spec=grid_spec,
        out_shape=jax.ShapeDtypeStruct((NP, D), jnp.float32),
        compiler_params=pltpu.CompilerParams(
            dimension_semantics=("arbitrary",)),
    )(sinfo, xs, ws.reshape(MAXNT, BT, 1), Wge, Wue, Wde)


def _shared_body(x_ref, wsg_ref, wsu_ref, wsd_ref, o_ref):
    k = pl.program_id(1)
    x = x_ref[...]
    g = lax.dot_general(x, wsg_ref[...], (((1,), (1,)), ((), ())),
                        preferred_element_type=jnp.float32)
    u = lax.dot_general(x, wsu_ref[...], (((1,), (1,)), ((), ())),
                        preferred_element_type=jnp.float32)
    h = g * lax.logistic(g) * u
    y = lax.dot_general(h, wsd_ref[...], (((1,), (1,)), ((), ())),
                        preferred_element_type=jnp.float32)

    @pl.when(k == 0)
    def _():
        o_ref[...] = y

    @pl.when(k != 0)
    def _():
        o_ref[...] += y


def _shared(xf, Wsg, Wsu, Wsd):
    RS, CS = 512, 512
    return pl.pallas_call(
        _shared_body,
        grid=(T // RS, SH // CS),
        in_specs=[
            pl.BlockSpec((RS, D), lambda i, k: (i, 0)),
            pl.BlockSpec((CS, D), lambda i, k: (k, 0)),
            pl.BlockSpec((CS, D), lambda i, k: (k, 0)),
            pl.BlockSpec((D, CS), lambda i, k: (0, k)),
        ],
        out_specs=pl.BlockSpec((RS, D), lambda i, k: (i, 0)),
        out_shape=jax.ShapeDtypeStruct((T, D), jnp.float32),
        compiler_params=pltpu.CompilerParams(
            dimension_semantics=("arbitrary", "arbitrary")),
    )(xf, Wsg, Wsu, Wsd)


def kernel(x, Wg, Wge, Wue, Wde, Wsg, Wsu, Wsd):
    b, s, d = x.shape
    xf = x.reshape(-1, d)

    pos, w, sinfo = _plan(xf, Wg)
    posf = pos.reshape(-1)
    wf = w.reshape(-1)
    sinfof = sinfo.reshape(-1)

    xs, ws = _sc_dispatch_kernel()(posf, wf, xf)
    shared = _shared(xf, Wsg, Wsu, Wsd)
    ys = _gemm(sinfof, xs, ws, Wge, Wue, Wde)
    out = _sc_combine_kernel()(ys, shared, posf)
    return out.reshape(b, s, d)


# speedup vs baseline: 1.2085x; 1.2068x over previous
import functools

import jax
import jax.numpy as jnp
from jax import lax
from jax.experimental import pallas as pl
from jax.experimental.pallas import tpu as pltpu
from jax.experimental.pallas import tpu_sc as plsc

T = 2048
D = 2048
DE = 1024
E = 8
SH = 2048
BT = 256
MAXNT = 24
NP = MAXNT * BT
NCH = 32
CH = 128
NW = 32


def _plan_body(x_ref, wg_ref, pos_ref, w_ref, sinfo_ref,
               cnt_ref, eid_s, rank_s):
    c = pl.program_id(0)

    @pl.when(c == 0)
    def _():
        cnt_ref[...] = jnp.zeros_like(cnt_ref)

    x = x_ref[...]
    logits = lax.dot_general(x, wg_ref[...], (((1,), (1,)), ((), ())),
                             preferred_element_type=jnp.float32)
    m = jnp.max(logits, axis=-1, keepdims=True)
    ex = jnp.exp(logits - m)
    sm = ex / jnp.sum(ex, axis=-1, keepdims=True)
    lane = lax.broadcasted_iota(jnp.int32, (CH, E), 1)
    i0 = jnp.argmax(sm, axis=-1)
    one0 = lane == i0[:, None]
    sm1 = jnp.where(one0, -jnp.inf, sm)
    i1 = jnp.argmax(sm1, axis=-1)
    k_is_1 = c >= (NCH // 2)
    idx = jnp.where(k_is_1, i1, i0).astype(jnp.int32)
    w = jnp.where(k_is_1, jnp.max(sm1, axis=-1), jnp.max(sm, axis=-1))

    oh = (lane == idx[:, None]).astype(jnp.float32)
    r_i = lax.broadcasted_iota(jnp.int32, (CH, CH), 0)
    c_i = lax.broadcasted_iota(jnp.int32, (CH, CH), 1)
    tri = (r_i > c_i).astype(jnp.float32)
    within = lax.dot_general(tri, oh, (((1,), (0,)), ((), ())),
                             preferred_element_type=jnp.float32)
    base = cnt_ref[...]
    rank = jnp.sum((within + base) * oh, axis=1)

    eid_s[pl.ds(c, 1), :] = idx.reshape(1, CH)
    rank_s[pl.ds(c, 1), :] = rank.astype(jnp.int32).reshape(1, CH)
    w_ref[...] = w.reshape(1, 1, CH)
    cnt_ref[...] = base + jnp.sum(oh, axis=0, keepdims=True)

    @pl.when(c == NCH - 1)
    def _():
        cnt = cnt_ref[...]
        nt = jnp.floor((cnt + (BT - 1.0)) * (1.0 / BT))
        er = lax.broadcasted_iota(jnp.int32, (E, E), 0)
        ec = lax.broadcasted_iota(jnp.int32, (E, E), 1)
        mu = (er < ec).astype(jnp.float32)
        cum_excl = lax.dot_general(nt, mu, (((1,), (0,)), ((), ())),
                                   preferred_element_type=jnp.float32)
        cum_incl = cum_excl + nt
        gstart = (cum_excl * BT).astype(jnp.int32)
        ntot = jnp.sum(nt).astype(jnp.int32).reshape(1, 1)
        ti = lax.broadcasted_iota(jnp.int32, (48, E), 0)
        texp = jnp.sum((ti >= cum_incl.astype(jnp.int32)).astype(jnp.int32),
                       axis=1)
        texp = jnp.minimum(texp, E - 1).reshape(1, 48)
        sinfo_ref[...] = jnp.concatenate(
            [texp, ntot, jnp.zeros((1, 7), jnp.int32), gstart], axis=1)
        eids = eid_s[...]
        poss = rank_s[...]
        for e in range(E):
            ge = lax.slice(gstart, (0, e), (1, e + 1))
            poss = poss + jnp.where(eids == e, ge, 0)
        pos_ref[...] = poss.reshape(NCH, 1, CH)


def _plan(xf, Wg):
    return pl.pallas_call(
        _plan_body,
        grid=(NCH,),
        in_specs=[
            pl.BlockSpec((CH, D), lambda c: (lax.rem(c, NCH // 2), 0)),
            pl.BlockSpec((E, D), lambda c: (0, 0)),
        ],
        out_specs=[
            pl.BlockSpec((NCH, 1, CH), lambda c: (0, 0, 0)),
            pl.BlockSpec((1, 1, CH), lambda c: (c, 0, 0)),
            pl.BlockSpec((1, 64), lambda c: (0, 0)),
        ],
        out_shape=[
            jax.ShapeDtypeStruct((NCH, 1, CH), jnp.int32),
            jax.ShapeDtypeStruct((NCH, 1, CH), jnp.float32),
            jax.ShapeDtypeStruct((1, 64), jnp.int32),
        ],
        scratch_shapes=[
            pltpu.VMEM((1, E), jnp.float32),
            pltpu.VMEM((NCH, CH), jnp.int32),
            pltpu.VMEM((NCH, CH), jnp.int32),
        ],
        compiler_params=pltpu.CompilerParams(
            dimension_semantics=("arbitrary",)),
    )(xf, Wg)


@functools.cache
def _sc_mesh():
    return plsc.VectorSubcoreMesh(core_axis_name="c", subcore_axis_name="s",
                                  num_cores=2, num_subcores=16)


@functools.cache
def _sc_dispatch_kernel():
    return functools.partial(
        pl.kernel,
        out_type=(jax.ShapeDtypeStruct((NP, D), jnp.float32),
                  jax.ShapeDtypeStruct((NP,), jnp.float32)),
        mesh=_sc_mesh(),
        scratch_types=[
            pltpu.VMEM((CH,), jnp.int32),
            pltpu.VMEM((CH,), jnp.float32),
            pltpu.VMEM((16, D), jnp.float32),
            pltpu.VMEM((16, D), jnp.float32),
            pltpu.SemaphoreType.DMA,
            pltpu.SemaphoreType.DMA,
            pltpu.SemaphoreType.DMA,
            pltpu.SemaphoreType.DMA,
            pltpu.SemaphoreType.DMA,
        ],
    )(_sc_dispatch_body)


def _sc_dispatch_body(pos_h, w_h, x_h, xs_h, ws_h,
                      pos_v, w_v, rows_a, rows_b, sem_a, sem_b,
                      sem_oa, sem_ob, sem_w):
    wid = lax.axis_index("s") * 2 + lax.axis_index("c")
    sbase = wid * CH
    tbase = lax.rem(wid, 16) * CH
    pltpu.sync_copy(pos_h.at[pl.ds(sbase, CH)], pos_v)
    pltpu.sync_copy(w_h.at[pl.ds(sbase, CH)], w_v)
    cp_w = pltpu.async_copy(w_v, ws_h.at[pos_v], sem_w)
    bufs = (rows_a, rows_b)
    sin = (sem_a, sem_b)
    sout = (sem_oa, sem_ob)
    nch = CH // 16
    cps_in = [None, None]
    cps_out = [None, None]
    cps_in[0] = pltpu.async_copy(x_h.at[pl.ds(tbase, 16)], bufs[0], sin[0])
    for i in range(nch):
        cur = i % 2
        nxt = 1 - cur
        if i + 1 < nch:
            if cps_out[nxt] is not None:
                cps_out[nxt].wait()
            cps_in[nxt] = pltpu.async_copy(
                x_h.at[pl.ds(tbase + (i + 1) * 16, 16)], bufs[nxt], sin[nxt])
        cps_in[cur].wait()
        pv = pos_v[pl.ds(i * 16, 16)]
        cps_out[cur] = pltpu.async_copy(bufs[cur], xs_h.at[pv], sout[cur])
    cp_w.wait()
    cps_out[0].wait()
    cps_out[1].wait()


TPW = T // NW


@functools.cache
def _sc_combine_kernel():
    return functools.partial(
        pl.kernel,
        out_type=jax.ShapeDtypeStruct((T, D), jnp.float32),
        mesh=_sc_mesh(),
        scratch_types=[
            pltpu.VMEM((TPW,), jnp.int32),
            pltpu.VMEM((TPW,), jnp.int32),
            pltpu.VMEM((16, D), jnp.float32),
            pltpu.VMEM((16, D), jnp.float32),
            pltpu.VMEM((16, D), jnp.float32),
            pltpu.SemaphoreType.DMA,
            pltpu.SemaphoreType.DMA,
            pltpu.SemaphoreType.DMA,
        ],
    )(_sc_combine_body)


def _sc_combine_body(ys_h, sh_h, pos_h, out_h,
                     p0_v, p1_v, a_v, b_v, s_v, sem_a, sem_b, sem_o):
    wid = lax.axis_index("s") * 2 + lax.axis_index("c")
    tbase = wid * TPW
    pltpu.sync_copy(pos_h.at[pl.ds(tbase, TPW)], p0_v)
    pltpu.sync_copy(pos_h.at[pl.ds(T + tbase, TPW)], p1_v)
    nch = TPW // 16
    cp_a = pltpu.async_copy(ys_h.at[p0_v[pl.ds(0, 16)]], a_v, sem_a)
    cp_b = pltpu.async_copy(ys_h.at[p1_v[pl.ds(0, 16)]], b_v, sem_b)
    cp_o = None
    for cc in range(nch):
        cp_a.wait()
        cp_b.wait()
        if cp_o is not None:
            cp_o.wait()
        pltpu.sync_copy(sh_h.at[pl.ds(tbase + cc * 16, 16)], s_v)

        for r in range(16):
            def _col(j, _, r=r):
                for u in range(4):
                    sl = pl.ds(j * 64 + u * 16, 16)
                    s_v[r, sl] = a_v[r, sl] + b_v[r, sl] + s_v[r, sl]
                return 0

            lax.fori_loop(0, D // 64, _col, 0)
        if cc + 1 < nch:
            cp_a = pltpu.async_copy(ys_h.at[p0_v[pl.ds((cc + 1) * 16, 16)]],
                                    a_v, sem_a)
            cp_b = pltpu.async_copy(ys_h.at[p1_v[pl.ds((cc + 1) * 16, 16)]],
                                    b_v, sem_b)
        cp_o = pltpu.async_copy(s_v, out_h.at[pl.ds(tbase + cc * 16, 16)],
                                sem_o)
    cp_o.wait()


def _gemm_body(sinfo_ref, xs_ref, ws_ref, wge_ref, wue_ref, wde_ref, ys_ref):
    i = pl.program_id(0)

    @pl.when(i < sinfo_ref[48])
    def _():
        xs = xs_ref[...]
        g = lax.dot_general(xs, wge_ref[0], (((1,), (1,)), ((), ())),
                            preferred_element_type=jnp.float32)
        u = lax.dot_general(xs, wue_ref[0], (((1,), (1,)), ((), ())),
                            preferred_element_type=jnp.float32)
        h = g * lax.logistic(g) * u * ws_ref[0]
        ys_ref[...] = lax.dot_general(h, wde_ref[0], (((1,), (1,)), ((), ())),
                                      preferred_element_type=jnp.float32)


def _gemm(sinfo, xs, ws, Wge, Wue, Wde):
    def _clamped(i, s):
        return (jnp.minimum(i, s[48] - 1), 0)

    def _clamped3(i, s):
        return (jnp.minimum(i, s[48] - 1), 0, 0)

    grid_spec = pltpu.PrefetchScalarGridSpec(
        num_scalar_prefetch=1,
        grid=(MAXNT,),
        in_specs=[
            pl.BlockSpec((BT, D), _clamped),
            pl.BlockSpec((1, BT, 1), _clamped3),
            pl.BlockSpec((1, DE, D), lambda i, s: (s[i], 0, 0)),
            pl.BlockSpec((1, DE, D), lambda i, s: (s[i], 0, 0)),
            pl.BlockSpec((1, D, DE), lambda i, s: (s[i], 0, 0)),
        ],
        out_specs=pl.BlockSpec((BT, D), _clamped),
    )
    return pl.pallas_call(
        _gemm_body,
        grid_spec=grid_spec,
        out_shape=jax.ShapeDtypeStruct((NP, D), jnp.float32),
        compiler_params=pltpu.CompilerParams(
            dimension_semantics=("arbitrary",)),
    )(sinfo, xs, ws.reshape(MAXNT, BT, 1), Wge, Wue, Wde)


def _shared_body(x_ref, wsg_ref, wsu_ref, wsd_ref, o_ref):
    k = pl.program_id(1)
    x = x_ref[...]
    g = lax.dot_general(x, wsg_ref[...], (((1,), (1,)), ((), ())),
                        preferred_element_type=jnp.float32)
    u = lax.dot_general(x, wsu_ref[...], (((1,), (1,)), ((), ())),
                        preferred_element_type=jnp.float32)
    h = g * lax.logistic(g) * u
    y = lax.dot_general(h, wsd_ref[...], (((1,), (1,)), ((), ())),
                        preferred_element_type=jnp.float32)

    @pl.when(k == 0)
    def _():
        o_ref[...] = y

    @pl.when(k != 0)
    def _():
        o_ref[...] += y


def _shared(xf, Wsg, Wsu, Wsd):
    RS, CS = 512, 512
    return pl.pallas_call(
        _shared_body,
        grid=(T // RS, SH // CS),
        in_specs=[
            pl.BlockSpec((RS, D), lambda i, k: (i, 0)),
            pl.BlockSpec((CS, D), lambda i, k: (k, 0)),
            pl.BlockSpec((CS, D), lambda i, k: (k, 0)),
            pl.BlockSpec((D, CS), lambda i, k: (0, k)),
        ],
        out_specs=pl.BlockSpec((RS, D), lambda i, k: (i, 0)),
        out_shape=jax.ShapeDtypeStruct((T, D), jnp.float32),
        compiler_params=pltpu.CompilerParams(
            dimension_semantics=("arbitrary", "arbitrary")),
    )(xf, Wsg, Wsu, Wsd)


def kernel(x, Wg, Wge, Wue, Wde, Wsg, Wsu, Wsd):
    b, s, d = x.shape
    xf = x.reshape(-1, d)

    pos, w, sinfo = _plan(xf, Wg)
    posf = pos.reshape(-1)
    wf = w.reshape(-1)
    sinfof = sinfo.reshape(-1)

    xs, ws = _sc_dispatch_kernel()(posf, wf, xf)
    shared = _shared(xf, Wsg, Wsu, Wsd)
    ys = _gemm(sinfof, xs, ws, Wge, Wue, Wde)
    out = _sc_combine_kernel()(ys, shared, posf)
    return out.reshape(b, s, d)
